# baseline (speedup 1.0000x reference)
import jax
import jax.numpy as jnp
from jax.experimental import pallas as pl
from jax.experimental.pallas import tpu as pltpu

_N_ROWS = 128
_N_COLS = 100000
_TILE = 2048
_GRID = (_N_COLS + _TILE - 1) // _TILE

_K0 = 0
_K1 = 42
_KS2 = _K0 ^ _K1 ^ 0x1BD11BDA
_TINY = float(jnp.finfo(jnp.float32).tiny)
_IMAX = 2**31 - 1


def _rotl(x, r):
    return (x << jnp.uint32(r)) | (x >> jnp.uint32(32 - r))


def _random_bits(x1):
    ks = (jnp.uint32(_K0), jnp.uint32(_K1), jnp.uint32(_KS2))
    rot_a = (13, 15, 26, 6)
    rot_b = (17, 29, 16, 24)
    x0 = jnp.zeros_like(x1) + ks[0]
    x1 = x1 + ks[1]
    for i in range(5):
        for r in rot_a if i % 2 == 0 else rot_b:
            x0 = x0 + x1
            x1 = _rotl(x1, r)
            x1 = x1 ^ x0
        x0 = x0 + ks[(i + 1) % 3]
        x1 = x1 + ks[(i + 2) % 3] + jnp.uint32(i + 1)
    return x0 ^ x1


def _gumbel_tile(col0, width=_TILE):
    rows = jax.lax.broadcasted_iota(jnp.uint32, (_N_ROWS, width), 0)
    cols = jax.lax.broadcasted_iota(jnp.uint32, (_N_ROWS, width), 1)
    flat = rows * jnp.uint32(_N_COLS) + cols + col0.astype(jnp.uint32)
    bits = _random_bits(flat)
    fl = jax.lax.bitcast_convert_type(
        (bits >> jnp.uint32(9)) | jnp.uint32(0x3F800000), jnp.float32
    ) - jnp.float32(1.0)
    tiny = jnp.float32(_TINY)
    u = jnp.maximum(tiny, fl * (jnp.float32(1.0) - tiny) + tiny)
    return -jnp.log(-jnp.log(u))


_LAST = _GRID - 1
_LAST_W = (_N_COLS - _LAST * _TILE) // 128 * 128
_TAIL0 = _LAST * _TILE + _LAST_W
_TAIL_W = _N_COLS - _TAIL0


def _start(logits_hbm, buf, sem, j, width):
    col = pl.multiple_of(j * _TILE, _TILE)
    slot = jax.lax.rem(j, 2)
    pltpu.make_async_copy(
        logits_hbm.at[:, pl.ds(col, width)],
        buf.at[slot, :, pl.ds(0, width)],
        sem.at[slot],
    ).start()


def _body(logits_hbm, out_ref, max_ref, idx_ref, buf, tail, sem):
    j = pl.program_id(0)
    slot = jax.lax.rem(j, 2)
    col0 = j * _TILE

    @pl.when(j == 0)
    def _():
        _start(logits_hbm, buf, sem, j, _TILE)
        pltpu.make_async_copy(
            logits_hbm.at[:, pl.ds(_TAIL0, _TAIL_W)], tail, sem.at[2]
        ).start()

    @pl.when(j + 1 < _LAST)
    def _():
        _start(logits_hbm, buf, sem, j + 1, _TILE)

    @pl.when(j + 1 == _LAST)
    def _():
        _start(logits_hbm, buf, sem, j + 1, _LAST_W)

    @pl.when(j < _LAST)
    def _():
        pltpu.make_async_copy(
            logits_hbm.at[:, pl.ds(0, _TILE)], buf.at[slot], sem.at[slot]
        ).wait()

    @pl.when(j == _LAST)
    def _():
        pltpu.make_async_copy(
            logits_hbm.at[:, pl.ds(0, _LAST_W)],
            buf.at[slot, :, pl.ds(0, _LAST_W)],
            sem.at[slot],
        ).wait()

    vals = buf[slot] + _gumbel_tile(col0)
    cids = jax.lax.broadcasted_iota(jnp.int32, (_N_ROWS, _TILE), 1) + col0
    limit = jnp.where(j == _LAST, col0 + _LAST_W, col0 + _TILE)
    vals = jnp.where(cids < limit, vals, -jnp.inf)

    m = jnp.max(vals, axis=1, keepdims=True)
    first = jnp.min(
        jnp.where(vals == m, cids, jnp.int32(_IMAX)), axis=1, keepdims=True
    )

    @pl.when(j == 0)
    def _():
        max_ref[...] = m
        idx_ref[...] = first

    @pl.when(j > 0)
    def _():
        better = m > max_ref[...]
        idx_ref[...] = jnp.where(better, first, idx_ref[...])
        max_ref[...] = jnp.where(better, m, max_ref[...])

    @pl.when(j == _GRID - 1)
    def _():
        pltpu.make_async_copy(
            logits_hbm.at[:, pl.ds(_TAIL0, _TAIL_W)], tail, sem.at[2]
        ).wait()
        tvals = tail[...] + _gumbel_tile(jnp.int32(_TAIL0), _TAIL_W)
        tcids = (
            jax.lax.broadcasted_iota(jnp.int32, (_N_ROWS, _TAIL_W), 1) + _TAIL0
        )
        tm = jnp.max(tvals, axis=1, keepdims=True)
        tfirst = jnp.min(
            jnp.where(tvals == tm, tcids, jnp.int32(_IMAX)),
            axis=1,
            keepdims=True,
        )
        better = tm > max_ref[...]
        out_ref[...] = jnp.where(better, tfirst, idx_ref[...])


def kernel(logits):
    out = pl.pallas_call(
        _body,
        grid=(_GRID,),
        in_specs=[pl.BlockSpec(memory_space=pl.ANY)],
        out_specs=pl.BlockSpec((_N_ROWS, 1), lambda j: (0, 0)),
        out_shape=jax.ShapeDtypeStruct((_N_ROWS, 1), jnp.int32),
        scratch_shapes=[
            pltpu.VMEM((_N_ROWS, 1), jnp.float32),
            pltpu.VMEM((_N_ROWS, 1), jnp.int32),
            pltpu.VMEM((2, _N_ROWS, _TILE), jnp.float32),
            pltpu.VMEM((_N_ROWS, _TAIL_W), jnp.float32),
            pltpu.SemaphoreType.DMA((3,)),
        ],
    )(logits)
    return out.astype(jnp.int64)


# speedup vs baseline: 1.0027x; 1.0027x over previous
import jax
import jax.numpy as jnp
from jax.experimental import pallas as pl
from jax.experimental.pallas import tpu as pltpu

_N_ROWS = 128
_N_COLS = 100000
_TILE = 2048
_GRID = (_N_COLS + _TILE - 1) // _TILE

_K0 = 0
_K1 = 42
_KS2 = _K0 ^ _K1 ^ 0x1BD11BDA
_TINY = float(jnp.finfo(jnp.float32).tiny)
_IMAX = 2**31 - 1


def _rotl(x, r):
    return (x << jnp.uint32(r)) | (x >> jnp.uint32(32 - r))


def _random_bits(x1):
    ks = (jnp.uint32(_K0), jnp.uint32(_K1), jnp.uint32(_KS2))
    rot_a = (13, 15, 26, 6)
    rot_b = (17, 29, 16, 24)
    x0 = jnp.zeros_like(x1) + ks[0]
    x1 = x1 + ks[1]
    for i in range(5):
        for r in rot_a if i % 2 == 0 else rot_b:
            x0 = x0 + x1
            x1 = _rotl(x1, r)
            x1 = x1 ^ x0
        x0 = x0 + ks[(i + 1) % 3]
        x1 = x1 + ks[(i + 2) % 3] + jnp.uint32(i + 1)
    return x0 ^ x1


def _gumbel_tile(col0):
    rows = jax.lax.broadcasted_iota(jnp.uint32, (_N_ROWS, _TILE), 0)
    cols = jax.lax.broadcasted_iota(jnp.uint32, (_N_ROWS, _TILE), 1)
    flat = rows * jnp.uint32(_N_COLS) + cols + col0.astype(jnp.uint32)
    bits = _random_bits(flat)
    fl = jax.lax.bitcast_convert_type(
        (bits >> jnp.uint32(9)) | jnp.uint32(0x3F800000), jnp.float32
    ) - jnp.float32(1.0)
    tiny = jnp.float32(_TINY)
    u = jnp.maximum(tiny, fl * (jnp.float32(1.0) - tiny) + tiny)
    return -jnp.log(-jnp.log(u))


def _body(logits_ref, out_ref, max_ref, idx_ref):
    j = pl.program_id(0)
    col0 = j * _TILE
    vals = logits_ref[...] + _gumbel_tile(col0)
    cids = jax.lax.broadcasted_iota(jnp.int32, (_N_ROWS, _TILE), 1) + col0
    vals = jnp.where(cids < _N_COLS, vals, -jnp.inf)

    m = jnp.max(vals, axis=1, keepdims=True)
    first = jnp.min(
        jnp.where(vals == m, cids, jnp.int32(_IMAX)), axis=1, keepdims=True
    )

    @pl.when(j == 0)
    def _():
        max_ref[...] = m
        idx_ref[...] = first

    @pl.when(j > 0)
    def _():
        better = m > max_ref[...]
        idx_ref[...] = jnp.where(better, first, idx_ref[...])
        max_ref[...] = jnp.where(better, m, max_ref[...])

    @pl.when(j == _GRID - 1)
    def _():
        out_ref[...] = idx_ref[...]


def kernel(logits):
    out = pl.pallas_call(
        _body,
        grid=(_GRID,),
        in_specs=[pl.BlockSpec((_N_ROWS, _TILE), lambda j: (0, j))],
        out_specs=pl.BlockSpec((_N_ROWS, 1), lambda j: (0, 0)),
        out_shape=jax.ShapeDtypeStruct((_N_ROWS, 1), jnp.int32),
        scratch_shapes=[
            pltpu.VMEM((_N_ROWS, 1), jnp.float32),
            pltpu.VMEM((_N_ROWS, 1), jnp.int32),
        ],
    )(logits)
    return out.astype(jnp.int64)
